# Initial kernel scaffold; baseline (speedup 1.0000x reference)
import jax
import jax.numpy as jnp
from jax.experimental import pallas as pl


def kernel(x, edge_index, batch, fc1_W, fc1_b, s1_Wl, s1_bl, s1_Wr, bn1_g, bn1_b, s2_Wl, s2_bl, s2_Wr, bn2_g, bn2_b, s3_Wl, s3_bl, s3_Wr, bn3_g, bn3_b, fc2_W, fc2_b, bn4_g, bn4_b, fc3_W, fc3_b, bn5_g, bn5_b, fc5_W, fc5_b, fc4_W, fc4_b):
    raise NotImplementedError("write your pallas kernel here")


# speedup vs baseline: 1.4058x; 1.4058x over previous
import functools

import jax
import jax.numpy as jnp
from jax import lax
from jax.experimental import pallas as pl
from jax.experimental.pallas import tpu as pltpu
from jax.experimental.pallas import tpu_sc as plsc

N = 10000
E = 160000
G = 64
NC, NS = 2, 16
NW = NC * NS
KB = 128
NB = 40
EP = NW * NB * KB
NPAD = 10240
RB = 1000
NBLK = N // RB
F32 = jnp.float32


def _fill2d(ref, val):
    rows, cols = ref.shape

    def row(i, carry):
        for j in range(cols // 16):
            ref[i, pl.ds(j * 16, 16)] = jnp.full((16,), val, ref.dtype)
        return carry

    lax.fori_loop(0, rows, row, 0)


def _sc_mesh():
    return plsc.VectorSubcoreMesh(
        core_axis_name="c", subcore_axis_name="s",
        num_cores=NC, num_subcores=NS)


def _make_segsum(C, with_deg):
    CT = C + (1 if with_deg else 0)
    out_type = [jax.ShapeDtypeStruct((NC, CT, NPAD, 128), F32)]

    scratch = [
        pltpu.VMEM((NB, KB), jnp.int32),
        pltpu.VMEM((NB, KB), jnp.int32),
        pltpu.VMEM((KB, 128), F32),
        pltpu.VMEM_SHARED((NPAD, 128), F32),
        pltpu.SemaphoreType.DMA,
    ]

    def body(src_hbm, dst_hbm, zh_hbm, *args):
        tables = args[:C]
        parts = args[C]
        (src_v, dst_v, rows, acc, sem) = args[C + 1:]

        ci = lax.axis_index("c")
        si = lax.axis_index("s")
        w = si * NC + ci

        pltpu.sync_copy(src_hbm.at[w], src_v)
        pltpu.sync_copy(dst_hbm.at[w], dst_v)

        for c in range(CT):
            plsc.subcore_barrier()
            pltpu.sync_copy(zh_hbm, acc.at[pl.ds(si * 640, 640)])
            if c == C:
                _fill2d(rows, 1.0)
            plsc.subcore_barrier()

            def batch(j, carry):
                if c < C:
                    pltpu.async_copy(tables[c].at[src_v.at[j]], rows, sem).wait()
                pltpu.sync_copy(rows, acc.at[dst_v.at[j]], add=True)
                return carry

            lax.fori_loop(0, NB, batch, 0)

            plsc.subcore_barrier()
            pltpu.sync_copy(acc.at[pl.ds(si * 640, 640)],
                            parts.at[ci, c, pl.ds(si * 640, 640)])

    return pl.kernel(body, out_type=out_type, mesh=_sc_mesh(),
                     scratch_types=scratch)


def _fc1_body(x_ref, w_ref, b_ref, out_ref):
    z = lax.dot_general(x_ref[...], w_ref[...], (((1,), (1,)), ((), ())),
                        preferred_element_type=F32)
    out_ref[...] = jnp.maximum(z + b_ref[...], 0.0)[None]


def _fc1(xp, wp, b2):
    return pl.pallas_call(
        _fc1_body,
        grid=(NBLK,),
        in_specs=[
            pl.BlockSpec((RB, 8), lambda i: (i, 0)),
            pl.BlockSpec((128, 8), lambda i: (0, 0)),
            pl.BlockSpec((1, 128), lambda i: (0, 0)),
        ],
        out_specs=pl.BlockSpec((1, RB, 128), lambda i: (0, i, 0)),
        out_shape=jax.ShapeDtypeStruct((1, N, 128), F32),
    )(xp, wp, b2)


def _invdeg_body(p_ref, out_ref):
    d = p_ref[0, 0, :, 0:1] + p_ref[1, 0, :, 0:1]
    out_ref[...] = jnp.broadcast_to(1.0 / jnp.maximum(d, 1.0), (RB, 16))


def _invdeg(parts1):
    return pl.pallas_call(
        _invdeg_body,
        grid=(NBLK,),
        in_specs=[pl.BlockSpec((NC, 1, RB, 128), lambda i: (0, 1, i, 0))],
        out_specs=pl.BlockSpec((RB, 16), lambda i: (i, 0)),
        out_shape=jax.ShapeDtypeStruct((N, 16), F32),
    )(parts1)


def _mm_stats_body(C, Co, has_agg, *refs):
    if has_agg:
        (parts, inv_ref, h, wl, wr, b, z_out, s_out, zacc, sacc) = refs
    else:
        (h, wl, b, z_out, s_out, zacc, sacc) = refs
    j = pl.program_id(0)
    i = pl.program_id(1)
    k = pl.program_id(2)

    hb = h[0]
    contrib = lax.dot_general(hb, wr[...] if has_agg else wl[...],
                              (((1,), (1,)), ((), ())),
                              preferred_element_type=F32)
    if has_agg:
        a = (parts[0, 0] + parts[1, 0]) * inv_ref[:, 0:1]
        contrib = contrib + lax.dot_general(a, wl[...], (((1,), (1,)), ((), ())),
                                            preferred_element_type=F32)

    @pl.when(k == 0)
    def _():
        zacc[...] = contrib

    @pl.when(k > 0)
    def _():
        zacc[...] = zacc[...] + contrib

    @pl.when(k == C - 1)
    def _():
        z = zacc[...] + b[0]
        z_out[...] = z[None]
        srow = jnp.sum(z, axis=0, keepdims=True)
        sqrow = jnp.sum(z * z, axis=0, keepdims=True)
        st = jnp.concatenate([srow, sqrow, jnp.zeros((6, 128), F32)], axis=0)

        @pl.when(i == 0)
        def _():
            sacc[...] = st

        @pl.when(i > 0)
        def _():
            sacc[...] = sacc[...] + st

        @pl.when(i == NBLK - 1)
        def _():
            s_out[...] = sacc[...][None]


def _sage_mm(parts, inv, h, wl, wr, b3):
    C = h.shape[0]
    Co = wl.shape[0] // 128
    body = functools.partial(_mm_stats_body, C, Co, True)
    return pl.pallas_call(
        body,
        grid=(Co, NBLK, C),
        in_specs=[
            pl.BlockSpec((NC, 1, RB, 128), lambda j, i, k: (0, k, i, 0)),
            pl.BlockSpec((RB, 16), lambda j, i, k: (i, 0)),
            pl.BlockSpec((1, RB, 128), lambda j, i, k: (k, i, 0)),
            pl.BlockSpec((128, 128), lambda j, i, k: (j, k)),
            pl.BlockSpec((128, 128), lambda j, i, k: (j, k)),
            pl.BlockSpec((1, 1, 128), lambda j, i, k: (j, 0, 0)),
        ],
        out_specs=[
            pl.BlockSpec((1, RB, 128), lambda j, i, k: (j, i, 0)),
            pl.BlockSpec((1, 8, 128), lambda j, i, k: (j, 0, 0)),
        ],
        out_shape=[
            jax.ShapeDtypeStruct((Co, N, 128), F32),
            jax.ShapeDtypeStruct((Co, 8, 128), F32),
        ],
        scratch_shapes=[
            pltpu.VMEM((RB, 128), F32),
            pltpu.VMEM((8, 128), F32),
        ],
    )(parts, inv, h, wl, wr, b3)


def _lin_mm(h, w, b3):
    C = h.shape[0]
    Co = w.shape[0] // 128
    body = functools.partial(_mm_stats_body, C, Co, False)
    return pl.pallas_call(
        body,
        grid=(Co, NBLK, C),
        in_specs=[
            pl.BlockSpec((1, RB, 128), lambda j, i, k: (k, i, 0)),
            pl.BlockSpec((128, 128), lambda j, i, k: (j, k)),
            pl.BlockSpec((1, 1, 128), lambda j, i, k: (j, 0, 0)),
        ],
        out_specs=[
            pl.BlockSpec((1, RB, 128), lambda j, i, k: (j, i, 0)),
            pl.BlockSpec((1, 8, 128), lambda j, i, k: (j, 0, 0)),
        ],
        out_shape=[
            jax.ShapeDtypeStruct((Co, N, 128), F32),
            jax.ShapeDtypeStruct((Co, 8, 128), F32),
        ],
        scratch_shapes=[
            pltpu.VMEM((RB, 128), F32),
            pltpu.VMEM((8, 128), F32),
        ],
    )(h, w, b3)


def _bn_scale_shift(stats, g, b):
    m = stats[0:1] / N
    var = stats[1:2] / N - m * m
    scale = g / jnp.sqrt(var + 1e-5)
    shift = b - m * scale
    return scale, shift


def _bn_relu_body(z_ref, s_ref, g_ref, b_ref, out_ref):
    scale, shift = _bn_scale_shift(s_ref[0], g_ref[0], b_ref[0])
    out_ref[...] = jnp.maximum(z_ref[0] * scale + shift, 0.0)[None]


def _bn_relu(z, s, g3, b3):
    Co = z.shape[0]
    return pl.pallas_call(
        _bn_relu_body,
        grid=(Co, NBLK),
        in_specs=[
            pl.BlockSpec((1, RB, 128), lambda j, i: (j, i, 0)),
            pl.BlockSpec((1, 8, 128), lambda j, i: (j, 0, 0)),
            pl.BlockSpec((1, 1, 128), lambda j, i: (j, 0, 0)),
            pl.BlockSpec((1, 1, 128), lambda j, i: (j, 0, 0)),
        ],
        out_specs=pl.BlockSpec((1, RB, 128), lambda j, i: (j, i, 0)),
        out_shape=jax.ShapeDtypeStruct((Co, N, 128), F32),
    )(z, s, g3, b3)


def _node_head_body(z_ref, s_ref, g_ref, b_ref, w5_ref, b5_ref, out_ref):
    acc = jnp.zeros((RB, 8), F32)
    for c in range(2):
        scale, shift = _bn_scale_shift(s_ref[c], g_ref[c], b_ref[c])
        node = jnp.maximum(z_ref[c] * scale + shift, 0.0)
        acc = acc + lax.dot_general(node, w5_ref[:, c * 128:(c + 1) * 128],
                                    (((1,), (1,)), ((), ())),
                                    preferred_element_type=F32)
    out_ref[...] = acc + b5_ref[...]


def _node_head(z, s, g3, b3, w5p, b5p):
    return pl.pallas_call(
        _node_head_body,
        grid=(NBLK,),
        in_specs=[
            pl.BlockSpec((2, RB, 128), lambda i: (0, i, 0)),
            pl.BlockSpec((2, 8, 128), lambda i: (0, 0, 0)),
            pl.BlockSpec((2, 1, 128), lambda i: (0, 0, 0)),
            pl.BlockSpec((2, 1, 128), lambda i: (0, 0, 0)),
            pl.BlockSpec((8, 256), lambda i: (0, 0)),
            pl.BlockSpec((1, 8), lambda i: (0, 0)),
        ],
        out_specs=pl.BlockSpec((RB, 8), lambda i: (i, 0)),
        out_shape=jax.ShapeDtypeStruct((N, 8), F32),
    )(z, s, g3, b3, w5p, b5p)


def _pool_head_body(h_ref, batch_ref, w4_ref, b4_ref, out_ref, gacc, cacc):
    i = pl.program_id(0)
    seg = batch_ref[...]
    gid = lax.broadcasted_iota(jnp.int32, (RB, 128), 1)
    p = jnp.where(gid == seg, 1.0, 0.0).astype(F32)

    ones = jnp.zeros((RB, 8), F32) + 1.0
    cnt = lax.dot_general(p, ones, (((0,), (0,)), ((), ())),
                          preferred_element_type=F32)

    @pl.when(i == 0)
    def _():
        cacc[...] = cnt
        for c in range(4):
            gacc[c] = lax.dot_general(p, h_ref[c], (((0,), (0,)), ((), ())),
                                      preferred_element_type=F32)

    @pl.when(i > 0)
    def _():
        cacc[...] = cacc[...] + cnt
        for c in range(4):
            gacc[c] = gacc[c] + lax.dot_general(
                p, h_ref[c], (((0,), (0,)), ((), ())),
                preferred_element_type=F32)

    @pl.when(i == NBLK - 1)
    def _():
        inv = 1.0 / jnp.maximum(cacc[...][:, 0:1], 1.0)
        acc = jnp.zeros((128, 8), F32)
        for c in range(4):
            gmean = gacc[c] * inv
            acc = acc + lax.dot_general(
                gmean, w4_ref[:, c * 128:(c + 1) * 128],
                (((1,), (1,)), ((), ())), preferred_element_type=F32)
        out_ref[...] = acc + b4_ref[...]


def _pool_head(h, batch2, w4p, b4p):
    return pl.pallas_call(
        _pool_head_body,
        grid=(NBLK,),
        in_specs=[
            pl.BlockSpec((4, RB, 128), lambda i: (0, i, 0)),
            pl.BlockSpec((RB, 1), lambda i: (i, 0)),
            pl.BlockSpec((8, 512), lambda i: (0, 0)),
            pl.BlockSpec((1, 8), lambda i: (0, 0)),
        ],
        out_specs=pl.BlockSpec((128, 8), lambda i: (0, 0)),
        out_shape=jax.ShapeDtypeStruct((128, 8), F32),
        scratch_shapes=[
            pltpu.VMEM((4, 128, 128), F32),
            pltpu.VMEM((128, 8), F32),
        ],
    )(h, batch2, w4p, b4p)


_sc_cache = {}


def _segsum1(srcp, dstp, zh, t0):
    if "s1" not in _sc_cache:
        _sc_cache["s1"] = _make_segsum(1, True)
    (parts,) = _sc_cache["s1"](srcp, dstp, zh, t0)
    return parts


def _segsum4(srcp, dstp, zh, *tables):
    if "s4" not in _sc_cache:
        _sc_cache["s4"] = _make_segsum(4, False)
    (parts,) = _sc_cache["s4"](srcp, dstp, zh, *tables)
    return parts


def _seg4(srcp, dstp, zh, h):
    return _segsum4(srcp, dstp, zh, h[0], h[1], h[2], h[3])


def kernel(x, edge_index, batch,
           fc1_W, fc1_b,
           s1_Wl, s1_bl, s1_Wr, bn1_g, bn1_b,
           s2_Wl, s2_bl, s2_Wr, bn2_g, bn2_b,
           s3_Wl, s3_bl, s3_Wr, bn3_g, bn3_b,
           fc2_W, fc2_b, bn4_g, bn4_b,
           fc3_W, fc3_b, bn5_g, bn5_b,
           fc5_W, fc5_b, fc4_W, fc4_b):
    src, dst = edge_index[0], edge_index[1]
    srcp = jnp.concatenate(
        [src, jnp.zeros((EP - E,), jnp.int32)]).reshape(NW, NB, KB)
    dstp = jnp.concatenate(
        [dst, jnp.full((EP - E,), N, jnp.int32)]).reshape(NW, NB, KB)

    xp = jnp.pad(x, ((0, 0), (0, 3)))
    w1p = jnp.pad(fc1_W, ((0, 0), (0, 3)))
    zh = jnp.zeros((640, 128), F32)

    h1 = _fc1(xp, w1p, fc1_b.reshape(1, 128))

    parts1 = _segsum1(srcp, dstp, zh, h1[0])
    inv = _invdeg(parts1)
    z1, s1 = _sage_mm(parts1, inv, h1, s1_Wl, s1_Wr, s1_bl.reshape(4, 1, 128))
    h2 = _bn_relu(z1, s1, bn1_g.reshape(4, 1, 128), bn1_b.reshape(4, 1, 128))

    parts2 = _seg4(srcp, dstp, zh, h2)
    z2, s2 = _sage_mm(parts2, inv, h2, s2_Wl, s2_Wr, s2_bl.reshape(4, 1, 128))
    h3 = _bn_relu(z2, s2, bn2_g.reshape(4, 1, 128), bn2_b.reshape(4, 1, 128))

    parts3 = _seg4(srcp, dstp, zh, h3)
    z3, s3 = _sage_mm(parts3, inv, h3, s3_Wl, s3_Wr, s3_bl.reshape(4, 1, 128))
    h3b = _bn_relu(z3, s3, bn3_g.reshape(4, 1, 128), bn3_b.reshape(4, 1, 128))

    z4, s4 = _lin_mm(h3b, fc2_W, fc2_b.reshape(4, 1, 128))
    h4 = _bn_relu(z4, s4, bn4_g.reshape(4, 1, 128), bn4_b.reshape(4, 1, 128))

    z5, s5 = _lin_mm(h4, fc3_W, fc3_b.reshape(2, 1, 128))
    w5p = jnp.pad(fc5_W, ((0, 3), (0, 0)))
    b5p = jnp.pad(fc5_b, (0, 3)).reshape(1, 8)
    node_out = _node_head(z5, s5, bn5_g.reshape(2, 1, 128),
                          bn5_b.reshape(2, 1, 128), w5p, b5p)[:, :5]

    w4p = jnp.pad(fc4_W, ((0, 5), (0, 0)))
    b4p = jnp.pad(fc4_b, (0, 5)).reshape(1, 8)
    graph_out = _pool_head(h4, batch.reshape(N, 1), w4p, b4p)[:G, :3]

    return (node_out, graph_out)


# speedup vs baseline: 1.4419x; 1.0256x over previous
import functools

import jax
import jax.numpy as jnp
from jax import lax
from jax.experimental import pallas as pl
from jax.experimental.pallas import tpu as pltpu
from jax.experimental.pallas import tpu_sc as plsc

N = 10000
E = 160000
G = 64
NC, NS = 2, 16
NW = NC * NS
KB = 64
NB = 80
EP = NW * NB * KB
NPAD = 10240
RB = 1000
NBLK = N // RB
F32 = jnp.float32


def _fill2d(ref, val):
    rows, cols = ref.shape

    def row(i, carry):
        for j in range(cols // 16):
            ref[i, pl.ds(j * 16, 16)] = jnp.full((16,), val, ref.dtype)
        return carry

    lax.fori_loop(0, rows, row, 0)


def _sc_mesh():
    return plsc.VectorSubcoreMesh(
        core_axis_name="c", subcore_axis_name="s",
        num_cores=NC, num_subcores=NS)


def _make_segsum(C, with_deg):
    CT = C + (1 if with_deg else 0)
    out_type = [jax.ShapeDtypeStruct((NC, CT, NPAD, 128), F32)]

    scratch = [
        pltpu.VMEM((NB, KB), jnp.int32),
        pltpu.VMEM((NB, KB), jnp.int32),
        pltpu.VMEM((KB, 128), F32),
        pltpu.VMEM((KB, 128), F32),
        pltpu.VMEM_SHARED((NPAD, 128), F32),
        pltpu.SemaphoreType.DMA,
        pltpu.SemaphoreType.DMA,
    ]

    def body(src_hbm, dst_hbm, zh_hbm, *args):
        tables = args[:C]
        parts = args[C]
        (src_v, dst_v, rows0, rows1, acc, sem0, sem1) = args[C + 1:]

        ci = lax.axis_index("c")
        si = lax.axis_index("s")
        w = si * NC + ci

        pltpu.sync_copy(src_hbm.at[w], src_v)
        pltpu.sync_copy(dst_hbm.at[w], dst_v)

        for c in range(CT):
            plsc.subcore_barrier()
            pltpu.sync_copy(zh_hbm, acc.at[pl.ds(si * 640, 640)])
            if c == C:
                _fill2d(rows0, 1.0)
            plsc.subcore_barrier()

            if c < C:
                tab = tables[c]

                def gather(j, buf, sem):
                    return pltpu.make_async_copy(tab.at[src_v.at[j]], buf, sem)

                gather(0, rows0, sem0).start()
                gather(1, rows1, sem1).start()

                def pair(j2, carry):
                    j = 2 * j2
                    gather(j, rows0, sem0).wait()
                    pltpu.sync_copy(rows0, acc.at[dst_v.at[j]], add=True)

                    @pl.when(j2 < NB // 2 - 1)
                    def _():
                        gather(j + 2, rows0, sem0).start()

                    gather(j + 1, rows1, sem1).wait()
                    pltpu.sync_copy(rows1, acc.at[dst_v.at[j + 1]], add=True)

                    @pl.when(j2 < NB // 2 - 1)
                    def _():
                        gather(j + 3, rows1, sem1).start()

                    return carry

                lax.fori_loop(0, NB // 2, pair, 0)
            else:
                def batch(j, carry):
                    pltpu.sync_copy(rows0, acc.at[dst_v.at[j]], add=True)
                    return carry

                lax.fori_loop(0, NB, batch, 0)

            plsc.subcore_barrier()
            pltpu.sync_copy(acc.at[pl.ds(si * 640, 640)],
                            parts.at[ci, c, pl.ds(si * 640, 640)])

    return pl.kernel(body, out_type=out_type, mesh=_sc_mesh(),
                     scratch_types=scratch)


def _fc1_body(x_ref, w_ref, b_ref, out_ref):
    z = lax.dot_general(x_ref[...], w_ref[...], (((1,), (1,)), ((), ())),
                        preferred_element_type=F32)
    out_ref[...] = jnp.maximum(z + b_ref[...], 0.0)[None]


def _fc1(xp, wp, b2):
    return pl.pallas_call(
        _fc1_body,
        grid=(NBLK,),
        in_specs=[
            pl.BlockSpec((RB, 8), lambda i: (i, 0)),
            pl.BlockSpec((128, 8), lambda i: (0, 0)),
            pl.BlockSpec((1, 128), lambda i: (0, 0)),
        ],
        out_specs=pl.BlockSpec((1, RB, 128), lambda i: (0, i, 0)),
        out_shape=jax.ShapeDtypeStruct((1, N, 128), F32),
    )(xp, wp, b2)


def _invdeg_body(p_ref, out_ref):
    d = p_ref[0, 0, :, 0:1] + p_ref[1, 0, :, 0:1]
    out_ref[...] = jnp.broadcast_to(1.0 / jnp.maximum(d, 1.0), (RB, 16))


def _invdeg(parts1):
    return pl.pallas_call(
        _invdeg_body,
        grid=(NBLK,),
        in_specs=[pl.BlockSpec((NC, 1, RB, 128), lambda i: (0, 1, i, 0))],
        out_specs=pl.BlockSpec((RB, 16), lambda i: (i, 0)),
        out_shape=jax.ShapeDtypeStruct((N, 16), F32),
    )(parts1)


def _mm_stats_body(C, Co, has_agg, *refs):
    if has_agg:
        (parts, inv_ref, h, wl, wr, b, z_out, s_out, zacc, sacc) = refs
    else:
        (h, wl, b, z_out, s_out, zacc, sacc) = refs
    j = pl.program_id(0)
    i = pl.program_id(1)
    k = pl.program_id(2)

    hb = h[0]
    contrib = lax.dot_general(hb, wr[...] if has_agg else wl[...],
                              (((1,), (1,)), ((), ())),
                              preferred_element_type=F32)
    if has_agg:
        a = (parts[0, 0] + parts[1, 0]) * inv_ref[:, 0:1]
        contrib = contrib + lax.dot_general(a, wl[...], (((1,), (1,)), ((), ())),
                                            preferred_element_type=F32)

    @pl.when(k == 0)
    def _():
        zacc[...] = contrib

    @pl.when(k > 0)
    def _():
        zacc[...] = zacc[...] + contrib

    @pl.when(k == C - 1)
    def _():
        z = zacc[...] + b[0]
        z_out[...] = z[None]
        srow = jnp.sum(z, axis=0, keepdims=True)
        sqrow = jnp.sum(z * z, axis=0, keepdims=True)
        st = jnp.concatenate([srow, sqrow, jnp.zeros((6, 128), F32)], axis=0)

        @pl.when(i == 0)
        def _():
            sacc[...] = st

        @pl.when(i > 0)
        def _():
            sacc[...] = sacc[...] + st

        @pl.when(i == NBLK - 1)
        def _():
            s_out[...] = sacc[...][None]


def _sage_mm(parts, inv, h, wl, wr, b3):
    C = h.shape[0]
    Co = wl.shape[0] // 128
    body = functools.partial(_mm_stats_body, C, Co, True)
    return pl.pallas_call(
        body,
        grid=(Co, NBLK, C),
        in_specs=[
            pl.BlockSpec((NC, 1, RB, 128), lambda j, i, k: (0, k, i, 0)),
            pl.BlockSpec((RB, 16), lambda j, i, k: (i, 0)),
            pl.BlockSpec((1, RB, 128), lambda j, i, k: (k, i, 0)),
            pl.BlockSpec((128, 128), lambda j, i, k: (j, k)),
            pl.BlockSpec((128, 128), lambda j, i, k: (j, k)),
            pl.BlockSpec((1, 1, 128), lambda j, i, k: (j, 0, 0)),
        ],
        out_specs=[
            pl.BlockSpec((1, RB, 128), lambda j, i, k: (j, i, 0)),
            pl.BlockSpec((1, 8, 128), lambda j, i, k: (j, 0, 0)),
        ],
        out_shape=[
            jax.ShapeDtypeStruct((Co, N, 128), F32),
            jax.ShapeDtypeStruct((Co, 8, 128), F32),
        ],
        scratch_shapes=[
            pltpu.VMEM((RB, 128), F32),
            pltpu.VMEM((8, 128), F32),
        ],
    )(parts, inv, h, wl, wr, b3)


def _lin_mm(h, w, b3):
    C = h.shape[0]
    Co = w.shape[0] // 128
    body = functools.partial(_mm_stats_body, C, Co, False)
    return pl.pallas_call(
        body,
        grid=(Co, NBLK, C),
        in_specs=[
            pl.BlockSpec((1, RB, 128), lambda j, i, k: (k, i, 0)),
            pl.BlockSpec((128, 128), lambda j, i, k: (j, k)),
            pl.BlockSpec((1, 1, 128), lambda j, i, k: (j, 0, 0)),
        ],
        out_specs=[
            pl.BlockSpec((1, RB, 128), lambda j, i, k: (j, i, 0)),
            pl.BlockSpec((1, 8, 128), lambda j, i, k: (j, 0, 0)),
        ],
        out_shape=[
            jax.ShapeDtypeStruct((Co, N, 128), F32),
            jax.ShapeDtypeStruct((Co, 8, 128), F32),
        ],
        scratch_shapes=[
            pltpu.VMEM((RB, 128), F32),
            pltpu.VMEM((8, 128), F32),
        ],
    )(h, w, b3)


def _bn_scale_shift(stats, g, b):
    m = stats[0:1] / N
    var = stats[1:2] / N - m * m
    scale = g / jnp.sqrt(var + 1e-5)
    shift = b - m * scale
    return scale, shift


def _bn_relu_body(z_ref, s_ref, g_ref, b_ref, out_ref):
    scale, shift = _bn_scale_shift(s_ref[0], g_ref[0], b_ref[0])
    out_ref[...] = jnp.maximum(z_ref[0] * scale + shift, 0.0)[None]


def _bn_relu(z, s, g3, b3):
    Co = z.shape[0]
    return pl.pallas_call(
        _bn_relu_body,
        grid=(Co, NBLK),
        in_specs=[
            pl.BlockSpec((1, RB, 128), lambda j, i: (j, i, 0)),
            pl.BlockSpec((1, 8, 128), lambda j, i: (j, 0, 0)),
            pl.BlockSpec((1, 1, 128), lambda j, i: (j, 0, 0)),
            pl.BlockSpec((1, 1, 128), lambda j, i: (j, 0, 0)),
        ],
        out_specs=pl.BlockSpec((1, RB, 128), lambda j, i: (j, i, 0)),
        out_shape=jax.ShapeDtypeStruct((Co, N, 128), F32),
    )(z, s, g3, b3)


def _node_head_body(z_ref, s_ref, g_ref, b_ref, w5_ref, b5_ref, out_ref):
    acc = jnp.zeros((RB, 8), F32)
    for c in range(2):
        scale, shift = _bn_scale_shift(s_ref[c], g_ref[c], b_ref[c])
        node = jnp.maximum(z_ref[c] * scale + shift, 0.0)
        acc = acc + lax.dot_general(node, w5_ref[:, c * 128:(c + 1) * 128],
                                    (((1,), (1,)), ((), ())),
                                    preferred_element_type=F32)
    out_ref[...] = acc + b5_ref[...]


def _node_head(z, s, g3, b3, w5p, b5p):
    return pl.pallas_call(
        _node_head_body,
        grid=(NBLK,),
        in_specs=[
            pl.BlockSpec((2, RB, 128), lambda i: (0, i, 0)),
            pl.BlockSpec((2, 8, 128), lambda i: (0, 0, 0)),
            pl.BlockSpec((2, 1, 128), lambda i: (0, 0, 0)),
            pl.BlockSpec((2, 1, 128), lambda i: (0, 0, 0)),
            pl.BlockSpec((8, 256), lambda i: (0, 0)),
            pl.BlockSpec((1, 8), lambda i: (0, 0)),
        ],
        out_specs=pl.BlockSpec((RB, 8), lambda i: (i, 0)),
        out_shape=jax.ShapeDtypeStruct((N, 8), F32),
    )(z, s, g3, b3, w5p, b5p)


def _pool_head_body(h_ref, batch_ref, w4_ref, b4_ref, out_ref, gacc, cacc):
    i = pl.program_id(0)
    seg = batch_ref[...]
    gid = lax.broadcasted_iota(jnp.int32, (RB, 128), 1)
    p = jnp.where(gid == seg, 1.0, 0.0).astype(F32)

    ones = jnp.zeros((RB, 8), F32) + 1.0
    cnt = lax.dot_general(p, ones, (((0,), (0,)), ((), ())),
                          preferred_element_type=F32)

    @pl.when(i == 0)
    def _():
        cacc[...] = cnt
        for c in range(4):
            gacc[c] = lax.dot_general(p, h_ref[c], (((0,), (0,)), ((), ())),
                                      preferred_element_type=F32)

    @pl.when(i > 0)
    def _():
        cacc[...] = cacc[...] + cnt
        for c in range(4):
            gacc[c] = gacc[c] + lax.dot_general(
                p, h_ref[c], (((0,), (0,)), ((), ())),
                preferred_element_type=F32)

    @pl.when(i == NBLK - 1)
    def _():
        inv = 1.0 / jnp.maximum(cacc[...][:, 0:1], 1.0)
        acc = jnp.zeros((128, 8), F32)
        for c in range(4):
            gmean = gacc[c] * inv
            acc = acc + lax.dot_general(
                gmean, w4_ref[:, c * 128:(c + 1) * 128],
                (((1,), (1,)), ((), ())), preferred_element_type=F32)
        out_ref[...] = acc + b4_ref[...]


def _pool_head(h, batch2, w4p, b4p):
    return pl.pallas_call(
        _pool_head_body,
        grid=(NBLK,),
        in_specs=[
            pl.BlockSpec((4, RB, 128), lambda i: (0, i, 0)),
            pl.BlockSpec((RB, 1), lambda i: (i, 0)),
            pl.BlockSpec((8, 512), lambda i: (0, 0)),
            pl.BlockSpec((1, 8), lambda i: (0, 0)),
        ],
        out_specs=pl.BlockSpec((128, 8), lambda i: (0, 0)),
        out_shape=jax.ShapeDtypeStruct((128, 8), F32),
        scratch_shapes=[
            pltpu.VMEM((4, 128, 128), F32),
            pltpu.VMEM((128, 8), F32),
        ],
    )(h, batch2, w4p, b4p)


_sc_cache = {}


def _segsum1(srcp, dstp, zh, t0):
    if "s1" not in _sc_cache:
        _sc_cache["s1"] = _make_segsum(1, True)
    (parts,) = _sc_cache["s1"](srcp, dstp, zh, t0)
    return parts


def _segsum4(srcp, dstp, zh, *tables):
    if "s4" not in _sc_cache:
        _sc_cache["s4"] = _make_segsum(4, False)
    (parts,) = _sc_cache["s4"](srcp, dstp, zh, *tables)
    return parts


def _seg4(srcp, dstp, zh, h):
    return _segsum4(srcp, dstp, zh, h[0], h[1], h[2], h[3])


def kernel(x, edge_index, batch,
           fc1_W, fc1_b,
           s1_Wl, s1_bl, s1_Wr, bn1_g, bn1_b,
           s2_Wl, s2_bl, s2_Wr, bn2_g, bn2_b,
           s3_Wl, s3_bl, s3_Wr, bn3_g, bn3_b,
           fc2_W, fc2_b, bn4_g, bn4_b,
           fc3_W, fc3_b, bn5_g, bn5_b,
           fc5_W, fc5_b, fc4_W, fc4_b):
    src, dst = edge_index[0], edge_index[1]
    srcp = jnp.concatenate(
        [src, jnp.zeros((EP - E,), jnp.int32)]).reshape(NW, NB, KB)
    dstp = jnp.concatenate(
        [dst, jnp.full((EP - E,), N, jnp.int32)]).reshape(NW, NB, KB)

    xp = jnp.pad(x, ((0, 0), (0, 3)))
    w1p = jnp.pad(fc1_W, ((0, 0), (0, 3)))
    zh = jnp.zeros((640, 128), F32)

    h1 = _fc1(xp, w1p, fc1_b.reshape(1, 128))

    parts1 = _segsum1(srcp, dstp, zh, h1[0])
    inv = _invdeg(parts1)
    z1, s1 = _sage_mm(parts1, inv, h1, s1_Wl, s1_Wr, s1_bl.reshape(4, 1, 128))
    h2 = _bn_relu(z1, s1, bn1_g.reshape(4, 1, 128), bn1_b.reshape(4, 1, 128))

    parts2 = _seg4(srcp, dstp, zh, h2)
    z2, s2 = _sage_mm(parts2, inv, h2, s2_Wl, s2_Wr, s2_bl.reshape(4, 1, 128))
    h3 = _bn_relu(z2, s2, bn2_g.reshape(4, 1, 128), bn2_b.reshape(4, 1, 128))

    parts3 = _seg4(srcp, dstp, zh, h3)
    z3, s3 = _sage_mm(parts3, inv, h3, s3_Wl, s3_Wr, s3_bl.reshape(4, 1, 128))
    h3b = _bn_relu(z3, s3, bn3_g.reshape(4, 1, 128), bn3_b.reshape(4, 1, 128))

    z4, s4 = _lin_mm(h3b, fc2_W, fc2_b.reshape(4, 1, 128))
    h4 = _bn_relu(z4, s4, bn4_g.reshape(4, 1, 128), bn4_b.reshape(4, 1, 128))

    z5, s5 = _lin_mm(h4, fc3_W, fc3_b.reshape(2, 1, 128))
    w5p = jnp.pad(fc5_W, ((0, 3), (0, 0)))
    b5p = jnp.pad(fc5_b, (0, 3)).reshape(1, 8)
    node_out = _node_head(z5, s5, bn5_g.reshape(2, 1, 128),
                          bn5_b.reshape(2, 1, 128), w5p, b5p)[:, :5]

    w4p = jnp.pad(fc4_W, ((0, 5), (0, 0)))
    b4p = jnp.pad(fc4_b, (0, 5)).reshape(1, 8)
    graph_out = _pool_head(h4, batch.reshape(N, 1), w4p, b4p)[:G, :3]

    return (node_out, graph_out)
